# baseline (speedup 1.0000x reference)
import functools

import jax
import jax.numpy as jnp
from jax import lax
from jax.experimental import pallas as pl
from jax.experimental.pallas import tpu as pltpu
from jax.experimental.pallas import tpu_sc as plsc

_NBATCH = 32
_NSAMPLES = 512
_DIM = 3
_NHID = 64
_NLAYERS = 3
_EPS = 1e-5
_N = _NBATCH * _NSAMPLES
_E = _N * 32

_NC = 2
_NS = 16
_NW = _NC * _NS
_NCHK = 8
_EC = _E // _NCHK
_EPWC = _EC // _NW
_CH = 512
_ITER = _EPWC // _CH
_IDXROWS = _CH // 128
_CHG = 256
_ITERG = _EPWC // _CHG
_IDXG = _CHG // 128
_ROWS_PER_SUBCORE = _N // _NS

_C0 = 3.0 ** (-_DIM)
_C1 = (2.0 ** (1 - _DIM)) / _NSAMPLES
_C2 = 1.0 / (_NSAMPLES * _NSAMPLES)

_F32 = jnp.float32
_SC_PARAMS = pltpu.CompilerParams(use_tc_tiling_on_sc=False)


def _sc_mesh():
    return plsc.VectorSubcoreMesh(core_axis_name="c", subcore_axis_name="s")


def _sc_gather(a_tab, b_tab, dstc, srcc):
    @functools.partial(
        pl.kernel,
        out_type=[
            jax.ShapeDtypeStruct((_EC, _NHID), _F32),
            jax.ShapeDtypeStruct((_EC, _NHID), _F32),
        ],
        mesh=_sc_mesh(),
        scratch_types=[
            pltpu.VMEM((2, _IDXG, 128), jnp.int32),
            pltpu.VMEM((2, _IDXG, 128), jnp.int32),
            pltpu.VMEM((2, _CHG, _NHID), _F32),
            pltpu.VMEM((2, _CHG, _NHID), _F32),
            pltpu.SemaphoreType.DMA,
            pltpu.SemaphoreType.DMA,
            pltpu.SemaphoreType.DMA,
            pltpu.SemaphoreType.DMA,
            pltpu.SemaphoreType.DMA,
        ],
        compiler_params=_SC_PARAMS,
    )
    def k(a_hbm, b_hbm, d_hbm, s_hbm, ag_hbm, bg_hbm,
          idxd, idxs, bufa, bufb, semg, si0, si1, so0, so1):
        wid = lax.axis_index("s") * _NC + lax.axis_index("c")
        si = (si0, si1)
        so = (so0, so1)

        def fire_idx(ci, b):
            r0 = wid * (_EPWC // 128) + ci * _IDXG
            pltpu.async_copy(d_hbm.at[pl.ds(r0, _IDXG)], idxd.at[b], si[b])
            pltpu.async_copy(s_hbm.at[pl.ds(r0, _IDXG)], idxs.at[b], si[b])

        def wait_idx(b):
            pltpu.make_async_copy(d_hbm.at[pl.ds(0, _IDXG)], idxd.at[b],
                                  si[b]).wait()
            pltpu.make_async_copy(s_hbm.at[pl.ds(0, _IDXG)], idxs.at[b],
                                  si[b]).wait()

        def fire_store(ci, b):
            e0 = wid * _EPWC + ci * _CHG
            pltpu.async_copy(bufa.at[b], ag_hbm.at[pl.ds(e0, _CHG)], so[b])
            pltpu.async_copy(bufb.at[b], bg_hbm.at[pl.ds(e0, _CHG)], so[b])

        def wait_store(b):
            pltpu.make_async_copy(bufa.at[b], ag_hbm.at[pl.ds(0, _CHG)],
                                  so[b]).wait()
            pltpu.make_async_copy(bufb.at[b], bg_hbm.at[pl.ds(0, _CHG)],
                                  so[b]).wait()

        fire_idx(0, 0)
        fire_idx(1, 1)

        @pl.loop(0, _ITERG, step=2)
        def _(g):
            for b in range(2):
                ci = g + b
                wait_idx(b)

                @pl.when(g > 0)
                def _():
                    wait_store(b)

                cps = []
                for j in range(_IDXG):
                    cps.append(pltpu.async_copy(
                        a_hbm.at[idxd.at[b, j]],
                        bufa.at[b].at[pl.ds(j * 128, 128)], semg))
                    cps.append(pltpu.async_copy(
                        b_hbm.at[idxs.at[b, j]],
                        bufb.at[b].at[pl.ds(j * 128, 128)], semg))
                for cp in cps:
                    cp.wait()

                @pl.when(ci + 2 < _ITERG)
                def _():
                    fire_idx(ci + 2, b)

                fire_store(ci, b)

        wait_store(0)
        wait_store(1)

    return k(a_tab, b_tab, dstc, srcc)


def _sc_scatter(msgs, dsts, zeros_tab):
    @functools.partial(
        pl.kernel,
        out_type=[
            jax.ShapeDtypeStruct((_N, _NHID), _F32),
            jax.ShapeDtypeStruct((_N, _NHID), _F32),
        ],
        mesh=_sc_mesh(),
        scratch_types=[
            pltpu.VMEM((2, _IDXG, 128), jnp.int32),
            pltpu.VMEM((2, _CHG, _NHID), _F32),
            pltpu.VMEM_SHARED((_N, _NHID), _F32),
            pltpu.SemaphoreType.DMA,
            pltpu.SemaphoreType.DMA,
        ],
        compiler_params=_SC_PARAMS,
    )
    def k(*refs):
        ms = refs[:_NCHK]
        ds_ = refs[_NCHK:2 * _NCHK]
        z_hbm, o0, o1, idx, val, accum, sl0, sl1 = refs[2 * _NCHK:]
        cid = lax.axis_index("c")
        sid = lax.axis_index("s")
        wid = sid * _NC + cid
        row0 = sid * _ROWS_PER_SUBCORE
        stripe = pl.ds(row0, _ROWS_PER_SUBCORE)
        sl = (sl0, sl1)
        pltpu.sync_copy(z_hbm.at[stripe], accum.at[stripe])
        plsc.subcore_barrier()

        for m_hbm, d_hbm in zip(ms, ds_):
            def fire_load(ci, b, m_hbm=m_hbm, d_hbm=d_hbm):
                e0 = wid * _EPWC + ci * _CHG
                r0 = wid * (_EPWC // 128) + ci * _IDXG
                pltpu.async_copy(d_hbm.at[pl.ds(r0, _IDXG)], idx.at[b],
                                 sl[b])
                pltpu.async_copy(m_hbm.at[pl.ds(e0, _CHG)], val.at[b], sl[b])

            def wait_load(b, m_hbm=m_hbm, d_hbm=d_hbm):
                pltpu.make_async_copy(d_hbm.at[pl.ds(0, _IDXG)],
                                      idx.at[b], sl[b]).wait()
                pltpu.make_async_copy(m_hbm.at[pl.ds(0, _CHG)], val.at[b],
                                      sl[b]).wait()

            fire_load(0, 0)
            fire_load(1, 1)

            @pl.loop(0, _ITERG, step=2)
            def _(g, fire_load=fire_load, wait_load=wait_load):
                for b in range(2):
                    ci = g + b
                    wait_load(b)

                    for j in range(_IDXG):
                        pltpu.sync_copy(val.at[b].at[pl.ds(j * 128, 128)],
                                        accum.at[idx.at[b, j]], add=True)

                    @pl.when(ci + 2 < _ITERG)
                    def _():
                        fire_load(ci + 2, b)

        plsc.subcore_barrier()

        @pl.when(cid == 0)
        def _():
            pltpu.sync_copy(accum.at[stripe], o0.at[stripe])

        @pl.when(cid == 1)
        def _():
            pltpu.sync_copy(accum.at[stripe], o1.at[stripe])

    return k(*msgs, *dsts, zeros_tab)


def _encode_body(x_ref, w_ref, b_ref, wa_ref, wb_ref, b1_ref,
                 h_ref, a_ref, bt_ref):
    h = (jnp.dot(x_ref[...], w_ref[...], preferred_element_type=_F32)
         + b_ref[...])
    h_ref[...] = h
    a_ref[...] = (jnp.dot(h, wa_ref[...], preferred_element_type=_F32)
                  + b1_ref[...])
    bt_ref[...] = jnp.dot(h, wb_ref[...], preferred_element_type=_F32)


def _edge_body(ag_ref, bg_ref, w2d_ref, b2d_ref, o_ref):
    s = jnp.maximum(ag_ref[...] + bg_ref[...], 0.0)
    o_ref[...] = jnp.maximum(
        jnp.dot(s, w2d_ref[...], preferred_element_type=_F32) + b2d_ref[...],
        0.0)


def _norm(u):
    mean = jnp.mean(u, axis=0, keepdims=True)
    d = u - mean
    var = jnp.mean(d * d, axis=0, keepdims=True)
    return d * lax.rsqrt(var + _EPS)


def _update_core(h_ref, p0_ref, p1_ref, w3h_ref, w3a_ref, b3_ref,
                 w4_ref, b4_ref):
    agg = p0_ref[...] + p1_ref[...]
    u = jnp.maximum(
        jnp.dot(h_ref[...], w3h_ref[...], preferred_element_type=_F32)
        + jnp.dot(agg, w3a_ref[...], preferred_element_type=_F32)
        + b3_ref[...], 0.0)
    u = jnp.maximum(
        jnp.dot(u, w4_ref[...], preferred_element_type=_F32) + b4_ref[...],
        0.0)
    return _norm(u)


def _update_ab_body(h_ref, p0_ref, p1_ref, w3h_ref, w3a_ref, b3_ref,
                    w4_ref, b4_ref, wa_ref, wb_ref, b1_ref,
                    o_ref, a_ref, bt_ref):
    hn = _update_core(h_ref, p0_ref, p1_ref, w3h_ref, w3a_ref, b3_ref,
                      w4_ref, b4_ref)
    o_ref[...] = hn
    a_ref[...] = (jnp.dot(hn, wa_ref[...], preferred_element_type=_F32)
                  + b1_ref[...])
    bt_ref[...] = jnp.dot(hn, wb_ref[...], preferred_element_type=_F32)


def _update_body(h_ref, p0_ref, p1_ref, w3h_ref, w3a_ref, b3_ref,
                 w4_ref, b4_ref, o_ref):
    o_ref[...] = _update_core(h_ref, p0_ref, p1_ref, w3h_ref, w3a_ref,
                              b3_ref, w4_ref, b4_ref)


def _decode_body(h_ref, w_ref, b_ref, o_ref):
    o_ref[...] = jax.nn.sigmoid(
        jnp.dot(h_ref[...], w_ref[...], preferred_element_type=_F32)
        + b_ref[...])


def _disc_body(x_ref, xt_ref, o_ref):
    x = x_ref[0]
    xt = xt_ref[0]
    c0, c1, c2 = x[:, 0:1], x[:, 1:2], x[:, 2:3]
    r0, r1, r2 = xt[0:1, :], xt[1:2, :], xt[2:3, :]
    p = ((1.0 - jnp.maximum(c0, r0))
         * (1.0 - jnp.maximum(c1, r1))
         * (1.0 - jnp.maximum(c2, r2)))
    sum2 = jnp.sum(p)
    prod1 = (1.0 - c0 * c0) * (1.0 - c1 * c1) * (1.0 - c2 * c2)
    sum1 = jnp.sum(prod1)
    val = jnp.sqrt(_C0 - _C1 * sum1 + _C2 * sum2)
    o_ref[...] = val * jnp.ones((1, 1, 128), _F32)


def _full(shape):
    return pl.BlockSpec(shape, lambda i: (0,) * len(shape))


def _rows(tile, width):
    return pl.BlockSpec((tile, width), lambda i: (i, 0))


_W = _full((_NHID, _NHID))
_BV = _full((1, _NHID))
_NSTRUCT = jax.ShapeDtypeStruct((_N, _NHID), _F32)


def _encode(X, enc_W, enc_b, wa, wb, b1):
    tile = 2048
    return pl.pallas_call(
        _encode_body,
        grid=(_N // tile,),
        in_specs=[_rows(tile, _DIM), _full((_DIM, _NHID)), _BV, _W, _W, _BV],
        out_specs=[_rows(tile, _NHID)] * 3,
        out_shape=[_NSTRUCT] * 3,
    )(X, enc_W, enc_b, wa, wb, b1)


def _edge_mlp(ag, bg, w2d, b2d):
    tile = 4096
    rows = _EC // 2
    out = pl.pallas_call(
        _edge_body,
        grid=(rows // tile,),
        in_specs=[_rows(tile, 128), _rows(tile, 128),
                  _full((128, 128)), _full((1, 128))],
        out_specs=_rows(tile, 128),
        out_shape=jax.ShapeDtypeStruct((rows, 128), _F32),
    )(ag.reshape(rows, 128), bg.reshape(rows, 128), w2d, b2d)
    return out.reshape(_EC, _NHID)


def _node_update_ab(h, p0, p1, w3h, w3a, b3, w4, b4, wa, wb, b1):
    return pl.pallas_call(
        _update_ab_body,
        grid=(_NBATCH,),
        in_specs=[_rows(_NSAMPLES, _NHID)] * 3 + [_W, _W, _BV, _W, _BV,
                                                  _W, _W, _BV],
        out_specs=[_rows(_NSAMPLES, _NHID)] * 3,
        out_shape=[_NSTRUCT] * 3,
    )(h, p0, p1, w3h, w3a, b3, w4, b4, wa, wb, b1)


def _node_update(h, p0, p1, w3h, w3a, b3, w4, b4):
    return pl.pallas_call(
        _update_body,
        grid=(_NBATCH,),
        in_specs=[_rows(_NSAMPLES, _NHID)] * 3 + [_W, _W, _BV, _W, _BV],
        out_specs=_rows(_NSAMPLES, _NHID),
        out_shape=_NSTRUCT,
    )(h, p0, p1, w3h, w3a, b3, w4, b4)


def _decode(h, dec_W, dec_b):
    tile = 2048
    return pl.pallas_call(
        _decode_body,
        grid=(_N // tile,),
        in_specs=[_rows(tile, _NHID), _full((_NHID, _DIM)), _full((1, _DIM))],
        out_specs=_rows(tile, _DIM),
        out_shape=jax.ShapeDtypeStruct((_N, _DIM), _F32),
    )(h, dec_W, dec_b)


def _discrepancy(Xo3, XoT):
    return pl.pallas_call(
        _disc_body,
        grid=(_NBATCH,),
        in_specs=[pl.BlockSpec((1, _NSAMPLES, _DIM), lambda b: (b, 0, 0)),
                  pl.BlockSpec((1, _DIM, _NSAMPLES), lambda b: (b, 0, 0))],
        out_specs=pl.BlockSpec((1, 1, 128), lambda b: (b, 0, 0)),
        out_shape=jax.ShapeDtypeStruct((_NBATCH, 1, 128), _F32),
    )(Xo3, XoT)


def _blockdiag(w2, b2):
    w2d = jnp.zeros((128, 128), _F32)
    w2d = w2d.at[:_NHID, :_NHID].set(w2).at[_NHID:, _NHID:].set(w2)
    return w2d, jnp.tile(b2, 2).reshape(1, 128)


def kernel(X, edge_index, batch, enc_W, enc_b, W1, b1, W2, b2, W3, b3,
           W4, b4, dec_W, dec_b):
    del batch
    src2d = edge_index[0].astype(jnp.int32).reshape(_E // 128, 128)
    dst2d = edge_index[1].astype(jnp.int32).reshape(_E // 128, 128)
    rows_c = _EC // 128
    srcc = [lax.slice_in_dim(src2d, c * rows_c, (c + 1) * rows_c)
            for c in range(_NCHK)]
    dstc = [lax.slice_in_dim(dst2d, c * rows_c, (c + 1) * rows_c)
            for c in range(_NCHK)]
    zeros_tab = jnp.zeros((_N, _NHID), _F32)

    h, a_tab, b_tab = _encode(X, enc_W, enc_b.reshape(1, _NHID),
                              W1[0, :_NHID], W1[0, _NHID:],
                              b1[0].reshape(1, _NHID))
    for l in range(_NLAYERS):
        w2d, b2d = _blockdiag(W2[l], b2[l])
        msgs = []
        for c in range(_NCHK):
            ag, bg = _sc_gather(a_tab, b_tab, dstc[c], srcc[c])
            msgs.append(_edge_mlp(ag, bg, w2d, b2d))
        p0, p1 = _sc_scatter(msgs, dstc, zeros_tab)
        if l + 1 < _NLAYERS:
            h, a_tab, b_tab = _node_update_ab(
                h, p0, p1, W3[l, :_NHID], W3[l, _NHID:],
                b3[l].reshape(1, _NHID), W4[l], b4[l].reshape(1, _NHID),
                W1[l + 1, :_NHID], W1[l + 1, _NHID:],
                b1[l + 1].reshape(1, _NHID))
        else:
            h = _node_update(h, p0, p1, W3[l, :_NHID], W3[l, _NHID:],
                             b3[l].reshape(1, _NHID), W4[l],
                             b4[l].reshape(1, _NHID))

    Xo = _decode(h, dec_W, dec_b.reshape(1, _DIM))
    Xo3 = Xo.reshape(_NBATCH, _NSAMPLES, _DIM)
    XoT = Xo3.transpose(0, 2, 1)
    disc = _discrepancy(Xo3, XoT)[:, 0, 0]
    loss = jnp.mean(disc)
    return (loss, Xo3)


# speedup vs baseline: 1.0356x; 1.0356x over previous
import functools

import jax
import jax.numpy as jnp
from jax import lax
from jax.experimental import pallas as pl
from jax.experimental.pallas import tpu as pltpu
from jax.experimental.pallas import tpu_sc as plsc

_NBATCH = 32
_NSAMPLES = 512
_DIM = 3
_NHID = 64
_NLAYERS = 3
_EPS = 1e-5
_N = _NBATCH * _NSAMPLES
_E = _N * 32

_NC = 2
_NS = 16
_NW = _NC * _NS
_NCHK = 4
_EC = _E // _NCHK
_EPWC = _EC // _NW
_CH = 512
_ITER = _EPWC // _CH
_IDXROWS = _CH // 128
_CHG = 256
_ITERG = _EPWC // _CHG
_IDXG = _CHG // 128
_ROWS_PER_SUBCORE = _N // _NS

_C0 = 3.0 ** (-_DIM)
_C1 = (2.0 ** (1 - _DIM)) / _NSAMPLES
_C2 = 1.0 / (_NSAMPLES * _NSAMPLES)

_F32 = jnp.float32
_SC_PARAMS = pltpu.CompilerParams(use_tc_tiling_on_sc=False)


def _sc_mesh():
    return plsc.VectorSubcoreMesh(core_axis_name="c", subcore_axis_name="s")


def _sc_gather(a_tab, b_tab, dstc, srcc):
    @functools.partial(
        pl.kernel,
        out_type=[
            jax.ShapeDtypeStruct((_EC, _NHID), _F32),
            jax.ShapeDtypeStruct((_EC, _NHID), _F32),
        ],
        mesh=_sc_mesh(),
        scratch_types=[
            pltpu.VMEM((2, _IDXG, 128), jnp.int32),
            pltpu.VMEM((2, _IDXG, 128), jnp.int32),
            pltpu.VMEM((2, _CHG, _NHID), _F32),
            pltpu.VMEM((2, _CHG, _NHID), _F32),
            pltpu.SemaphoreType.DMA,
            pltpu.SemaphoreType.DMA,
            pltpu.SemaphoreType.DMA,
            pltpu.SemaphoreType.DMA,
            pltpu.SemaphoreType.DMA,
        ],
        compiler_params=_SC_PARAMS,
    )
    def k(a_hbm, b_hbm, d_hbm, s_hbm, ag_hbm, bg_hbm,
          idxd, idxs, bufa, bufb, semg, si0, si1, so0, so1):
        wid = lax.axis_index("s") * _NC + lax.axis_index("c")
        si = (si0, si1)
        so = (so0, so1)

        def fire_idx(ci, b):
            r0 = wid * (_EPWC // 128) + ci * _IDXG
            pltpu.async_copy(d_hbm.at[pl.ds(r0, _IDXG)], idxd.at[b], si[b])
            pltpu.async_copy(s_hbm.at[pl.ds(r0, _IDXG)], idxs.at[b], si[b])

        def wait_idx(b):
            pltpu.make_async_copy(d_hbm.at[pl.ds(0, _IDXG)], idxd.at[b],
                                  si[b]).wait()
            pltpu.make_async_copy(s_hbm.at[pl.ds(0, _IDXG)], idxs.at[b],
                                  si[b]).wait()

        def fire_store(ci, b):
            e0 = wid * _EPWC + ci * _CHG
            pltpu.async_copy(bufa.at[b], ag_hbm.at[pl.ds(e0, _CHG)], so[b])
            pltpu.async_copy(bufb.at[b], bg_hbm.at[pl.ds(e0, _CHG)], so[b])

        def wait_store(b):
            pltpu.make_async_copy(bufa.at[b], ag_hbm.at[pl.ds(0, _CHG)],
                                  so[b]).wait()
            pltpu.make_async_copy(bufb.at[b], bg_hbm.at[pl.ds(0, _CHG)],
                                  so[b]).wait()

        fire_idx(0, 0)
        fire_idx(1, 1)

        @pl.loop(0, _ITERG, step=2)
        def _(g):
            for b in range(2):
                ci = g + b
                wait_idx(b)

                @pl.when(g > 0)
                def _():
                    wait_store(b)

                cps = []
                for j in range(_IDXG):
                    cps.append(pltpu.async_copy(
                        a_hbm.at[idxd.at[b, j]],
                        bufa.at[b].at[pl.ds(j * 128, 128)], semg))
                    cps.append(pltpu.async_copy(
                        b_hbm.at[idxs.at[b, j]],
                        bufb.at[b].at[pl.ds(j * 128, 128)], semg))
                for cp in cps:
                    cp.wait()

                @pl.when(ci + 2 < _ITERG)
                def _():
                    fire_idx(ci + 2, b)

                fire_store(ci, b)

        wait_store(0)
        wait_store(1)

    return k(a_tab, b_tab, dstc, srcc)


def _sc_scatter(msgs, dsts, zeros_tab):
    @functools.partial(
        pl.kernel,
        out_type=[
            jax.ShapeDtypeStruct((_N, _NHID), _F32),
            jax.ShapeDtypeStruct((_N, _NHID), _F32),
        ],
        mesh=_sc_mesh(),
        scratch_types=[
            pltpu.VMEM((2, _IDXG, 128), jnp.int32),
            pltpu.VMEM((2, _CHG, _NHID), _F32),
            pltpu.VMEM_SHARED((_N, _NHID), _F32),
            pltpu.SemaphoreType.DMA,
            pltpu.SemaphoreType.DMA,
        ],
        compiler_params=_SC_PARAMS,
    )
    def k(*refs):
        ms = refs[:_NCHK]
        ds_ = refs[_NCHK:2 * _NCHK]
        z_hbm, o0, o1, idx, val, accum, sl0, sl1 = refs[2 * _NCHK:]
        cid = lax.axis_index("c")
        sid = lax.axis_index("s")
        wid = sid * _NC + cid
        row0 = sid * _ROWS_PER_SUBCORE
        stripe = pl.ds(row0, _ROWS_PER_SUBCORE)
        sl = (sl0, sl1)
        pltpu.sync_copy(z_hbm.at[stripe], accum.at[stripe])
        plsc.subcore_barrier()

        for m_hbm, d_hbm in zip(ms, ds_):
            def fire_load(ci, b, m_hbm=m_hbm, d_hbm=d_hbm):
                e0 = wid * _EPWC + ci * _CHG
                r0 = wid * (_EPWC // 128) + ci * _IDXG
                pltpu.async_copy(d_hbm.at[pl.ds(r0, _IDXG)], idx.at[b],
                                 sl[b])
                pltpu.async_copy(m_hbm.at[pl.ds(e0, _CHG)], val.at[b], sl[b])

            def wait_load(b, m_hbm=m_hbm, d_hbm=d_hbm):
                pltpu.make_async_copy(d_hbm.at[pl.ds(0, _IDXG)],
                                      idx.at[b], sl[b]).wait()
                pltpu.make_async_copy(m_hbm.at[pl.ds(0, _CHG)], val.at[b],
                                      sl[b]).wait()

            fire_load(0, 0)
            fire_load(1, 1)

            @pl.loop(0, _ITERG, step=2)
            def _(g, fire_load=fire_load, wait_load=wait_load):
                for b in range(2):
                    ci = g + b
                    wait_load(b)

                    adds = []
                    for j in range(_IDXG):
                        adds.append(pltpu.async_copy(
                            val.at[b].at[pl.ds(j * 128, 128)],
                            accum.at[idx.at[b, j]], sl[b], add=True))
                    for cp in adds:
                        cp.wait()

                    @pl.when(ci + 2 < _ITERG)
                    def _():
                        fire_load(ci + 2, b)

        plsc.subcore_barrier()

        @pl.when(cid == 0)
        def _():
            pltpu.sync_copy(accum.at[stripe], o0.at[stripe])

        @pl.when(cid == 1)
        def _():
            pltpu.sync_copy(accum.at[stripe], o1.at[stripe])

    return k(*msgs, *dsts, zeros_tab)


def _encode_body(x_ref, w_ref, b_ref, wa_ref, wb_ref, b1_ref,
                 h_ref, a_ref, bt_ref):
    h = (jnp.dot(x_ref[...], w_ref[...], preferred_element_type=_F32)
         + b_ref[...])
    h_ref[...] = h
    a_ref[...] = (jnp.dot(h, wa_ref[...], preferred_element_type=_F32)
                  + b1_ref[...])
    bt_ref[...] = jnp.dot(h, wb_ref[...], preferred_element_type=_F32)


def _edge_body(ag_ref, bg_ref, w2d_ref, b2d_ref, o_ref):
    s = jnp.maximum(ag_ref[...] + bg_ref[...], 0.0)
    o_ref[...] = jnp.maximum(
        jnp.dot(s, w2d_ref[...], preferred_element_type=_F32) + b2d_ref[...],
        0.0)


def _norm(u):
    mean = jnp.mean(u, axis=0, keepdims=True)
    d = u - mean
    var = jnp.mean(d * d, axis=0, keepdims=True)
    return d * lax.rsqrt(var + _EPS)


def _update_core(h_ref, p0_ref, p1_ref, w3h_ref, w3a_ref, b3_ref,
                 w4_ref, b4_ref):
    agg = p0_ref[...] + p1_ref[...]
    u = jnp.maximum(
        jnp.dot(h_ref[...], w3h_ref[...], preferred_element_type=_F32)
        + jnp.dot(agg, w3a_ref[...], preferred_element_type=_F32)
        + b3_ref[...], 0.0)
    u = jnp.maximum(
        jnp.dot(u, w4_ref[...], preferred_element_type=_F32) + b4_ref[...],
        0.0)
    return _norm(u)


def _update_ab_body(h_ref, p0_ref, p1_ref, w3h_ref, w3a_ref, b3_ref,
                    w4_ref, b4_ref, wa_ref, wb_ref, b1_ref,
                    o_ref, a_ref, bt_ref):
    hn = _update_core(h_ref, p0_ref, p1_ref, w3h_ref, w3a_ref, b3_ref,
                      w4_ref, b4_ref)
    o_ref[...] = hn
    a_ref[...] = (jnp.dot(hn, wa_ref[...], preferred_element_type=_F32)
                  + b1_ref[...])
    bt_ref[...] = jnp.dot(hn, wb_ref[...], preferred_element_type=_F32)


def _update_body(h_ref, p0_ref, p1_ref, w3h_ref, w3a_ref, b3_ref,
                 w4_ref, b4_ref, o_ref):
    o_ref[...] = _update_core(h_ref, p0_ref, p1_ref, w3h_ref, w3a_ref,
                              b3_ref, w4_ref, b4_ref)


def _decode_body(h_ref, w_ref, b_ref, o_ref):
    o_ref[...] = jax.nn.sigmoid(
        jnp.dot(h_ref[...], w_ref[...], preferred_element_type=_F32)
        + b_ref[...])


def _disc_body(x_ref, xt_ref, o_ref):
    x = x_ref[0]
    xt = xt_ref[0]
    c0, c1, c2 = x[:, 0:1], x[:, 1:2], x[:, 2:3]
    r0, r1, r2 = xt[0:1, :], xt[1:2, :], xt[2:3, :]
    p = ((1.0 - jnp.maximum(c0, r0))
         * (1.0 - jnp.maximum(c1, r1))
         * (1.0 - jnp.maximum(c2, r2)))
    sum2 = jnp.sum(p)
    prod1 = (1.0 - c0 * c0) * (1.0 - c1 * c1) * (1.0 - c2 * c2)
    sum1 = jnp.sum(prod1)
    val = jnp.sqrt(_C0 - _C1 * sum1 + _C2 * sum2)
    o_ref[...] = val * jnp.ones((1, 1, 128), _F32)


def _full(shape):
    return pl.BlockSpec(shape, lambda i: (0,) * len(shape))


def _rows(tile, width):
    return pl.BlockSpec((tile, width), lambda i: (i, 0))


_W = _full((_NHID, _NHID))
_BV = _full((1, _NHID))
_NSTRUCT = jax.ShapeDtypeStruct((_N, _NHID), _F32)


def _encode(X, enc_W, enc_b, wa, wb, b1):
    tile = 2048
    return pl.pallas_call(
        _encode_body,
        grid=(_N // tile,),
        in_specs=[_rows(tile, _DIM), _full((_DIM, _NHID)), _BV, _W, _W, _BV],
        out_specs=[_rows(tile, _NHID)] * 3,
        out_shape=[_NSTRUCT] * 3,
    )(X, enc_W, enc_b, wa, wb, b1)


def _edge_mlp(ag, bg, w2d, b2d):
    tile = 4096
    rows = _EC // 2
    out = pl.pallas_call(
        _edge_body,
        grid=(rows // tile,),
        in_specs=[_rows(tile, 128), _rows(tile, 128),
                  _full((128, 128)), _full((1, 128))],
        out_specs=_rows(tile, 128),
        out_shape=jax.ShapeDtypeStruct((rows, 128), _F32),
    )(ag.reshape(rows, 128), bg.reshape(rows, 128), w2d, b2d)
    return out.reshape(_EC, _NHID)


def _node_update_ab(h, p0, p1, w3h, w3a, b3, w4, b4, wa, wb, b1):
    return pl.pallas_call(
        _update_ab_body,
        grid=(_NBATCH,),
        in_specs=[_rows(_NSAMPLES, _NHID)] * 3 + [_W, _W, _BV, _W, _BV,
                                                  _W, _W, _BV],
        out_specs=[_rows(_NSAMPLES, _NHID)] * 3,
        out_shape=[_NSTRUCT] * 3,
    )(h, p0, p1, w3h, w3a, b3, w4, b4, wa, wb, b1)


def _node_update(h, p0, p1, w3h, w3a, b3, w4, b4):
    return pl.pallas_call(
        _update_body,
        grid=(_NBATCH,),
        in_specs=[_rows(_NSAMPLES, _NHID)] * 3 + [_W, _W, _BV, _W, _BV],
        out_specs=_rows(_NSAMPLES, _NHID),
        out_shape=_NSTRUCT,
    )(h, p0, p1, w3h, w3a, b3, w4, b4)


def _decode(h, dec_W, dec_b):
    tile = 2048
    return pl.pallas_call(
        _decode_body,
        grid=(_N // tile,),
        in_specs=[_rows(tile, _NHID), _full((_NHID, _DIM)), _full((1, _DIM))],
        out_specs=_rows(tile, _DIM),
        out_shape=jax.ShapeDtypeStruct((_N, _DIM), _F32),
    )(h, dec_W, dec_b)


def _discrepancy(Xo3, XoT):
    return pl.pallas_call(
        _disc_body,
        grid=(_NBATCH,),
        in_specs=[pl.BlockSpec((1, _NSAMPLES, _DIM), lambda b: (b, 0, 0)),
                  pl.BlockSpec((1, _DIM, _NSAMPLES), lambda b: (b, 0, 0))],
        out_specs=pl.BlockSpec((1, 1, 128), lambda b: (b, 0, 0)),
        out_shape=jax.ShapeDtypeStruct((_NBATCH, 1, 128), _F32),
    )(Xo3, XoT)


def _blockdiag(w2, b2):
    w2d = jnp.zeros((128, 128), _F32)
    w2d = w2d.at[:_NHID, :_NHID].set(w2).at[_NHID:, _NHID:].set(w2)
    return w2d, jnp.tile(b2, 2).reshape(1, 128)


def kernel(X, edge_index, batch, enc_W, enc_b, W1, b1, W2, b2, W3, b3,
           W4, b4, dec_W, dec_b):
    del batch
    src2d = edge_index[0].astype(jnp.int32).reshape(_E // 128, 128)
    dst2d = edge_index[1].astype(jnp.int32).reshape(_E // 128, 128)
    rows_c = _EC // 128
    srcc = [lax.slice_in_dim(src2d, c * rows_c, (c + 1) * rows_c)
            for c in range(_NCHK)]
    dstc = [lax.slice_in_dim(dst2d, c * rows_c, (c + 1) * rows_c)
            for c in range(_NCHK)]
    zeros_tab = jnp.zeros((_N, _NHID), _F32)

    h, a_tab, b_tab = _encode(X, enc_W, enc_b.reshape(1, _NHID),
                              W1[0, :_NHID], W1[0, _NHID:],
                              b1[0].reshape(1, _NHID))
    for l in range(_NLAYERS):
        w2d, b2d = _blockdiag(W2[l], b2[l])
        msgs = []
        for c in range(_NCHK):
            ag, bg = _sc_gather(a_tab, b_tab, dstc[c], srcc[c])
            msgs.append(_edge_mlp(ag, bg, w2d, b2d))
        p0, p1 = _sc_scatter(msgs, dstc, zeros_tab)
        if l + 1 < _NLAYERS:
            h, a_tab, b_tab = _node_update_ab(
                h, p0, p1, W3[l, :_NHID], W3[l, _NHID:],
                b3[l].reshape(1, _NHID), W4[l], b4[l].reshape(1, _NHID),
                W1[l + 1, :_NHID], W1[l + 1, _NHID:],
                b1[l + 1].reshape(1, _NHID))
        else:
            h = _node_update(h, p0, p1, W3[l, :_NHID], W3[l, _NHID:],
                             b3[l].reshape(1, _NHID), W4[l],
                             b4[l].reshape(1, _NHID))

    Xo = _decode(h, dec_W, dec_b.reshape(1, _DIM))
    Xo3 = Xo.reshape(_NBATCH, _NSAMPLES, _DIM)
    XoT = Xo3.transpose(0, 2, 1)
    disc = _discrepancy(Xo3, XoT)[:, 0, 0]
    loss = jnp.mean(disc)
    return (loss, Xo3)
